# baseline (speedup 1.0000x reference)
import functools

import jax
import jax.numpy as jnp
from jax import lax
from jax.experimental import pallas as pl
from jax.experimental.pallas import tpu as pltpu
from jax.experimental.pallas import tpu_sc as plsc

B = 16384
NUM_BINS = 10
D = 64
EPS = 1e-5

NC = 2
NS = 16
NW = NC * NS
BPW = B // NW
IC = 128
NCHUNK = BPW // IC


@functools.lru_cache(maxsize=None)
def _get_sc_gather():
    mesh = plsc.VectorSubcoreMesh(core_axis_name="c", subcore_axis_name="s",
                                  num_cores=NC, num_subcores=NS)

    @functools.partial(
        pl.kernel,
        mesh=mesh,
        out_type=jax.ShapeDtypeStruct((B, D), jnp.float32),
        scratch_types=[
            pltpu.VMEM((NCHUNK, IC), jnp.int32),
            pltpu.VMEM((BPW, D), jnp.float32),
            pltpu.SemaphoreType.DMA,
        ],
        compiler_params=pltpu.CompilerParams(use_tc_tiling_on_sc=False),
    )
    def _sc_gather(table, idx, out, idx_v, rows_v, sem):
        wid = lax.axis_index("s") * NC + lax.axis_index("c")
        base = wid * BPW
        pltpu.sync_copy(idx.at[pl.ds(wid * NCHUNK, NCHUNK)], idx_v)
        for j in range(NCHUNK):
            pltpu.async_copy(table.at[idx_v.at[j]], rows_v.at[pl.ds(j * IC, IC)], sem)
        for j in range(NCHUNK):
            pltpu.make_async_copy(table.at[idx_v.at[j]], rows_v.at[pl.ds(j * IC, IC)], sem).wait()
        pltpu.sync_copy(rows_v, out.at[pl.ds(base, BPW)])

    return _sc_gather


def _layernorm(tok, mmat, gamma, beta):
    mu = jnp.dot(tok, mmat, preferred_element_type=jnp.float32)
    xc = tok - mu
    var = jnp.dot(xc * xc, mmat, preferred_element_type=jnp.float32)
    return xc * lax.rsqrt(var + EPS) * gamma + beta


def _tc_body(nums_ref, g0_ref, g1_ref, g2_ref, g3_ref, centers_ref, w_ref,
             bias_ref, gamma_ref, beta_ref, out_ref):
    gamma = gamma_ref[0:1, :]
    beta = beta_ref[0:1, :]
    mmat = jnp.full((D, D), 1.0 / D, dtype=jnp.float32)
    for f in range(4):
        x = nums_ref[:, f:f + 1]
        mask = jnp.isnan(x)
        clean = jnp.where(mask, 0.0, x)
        d = -((clean - centers_ref[f:f + 1, :]) ** 2)
        d = d - jnp.max(d, axis=-1, keepdims=True)
        e = jnp.exp(d)
        p = e / jnp.sum(e, axis=-1, keepdims=True)
        tok = lax.dot_general(p, w_ref[f], (((1,), (1,)), ((), ())),
                              preferred_element_type=jnp.float32)
        tok = tok + bias_ref[f:f + 1, :]
        tok = jnp.where(mask, 0.0, tok)
        out_ref[:, f, :] = _layernorm(tok, mmat, gamma, beta)
    for f, g_ref in enumerate((g0_ref, g1_ref, g2_ref, g3_ref)):
        out_ref[:, 4 + f, :] = _layernorm(g_ref[...], mmat, gamma, beta)


BM = 2048


def _tc_call(nums, gs, centers, w, bias, gamma, beta, interpret=False):
    grid = B // BM
    gspec = pl.BlockSpec((BM, D), lambda i: (i, 0))
    return pl.pallas_call(
        _tc_body,
        grid=(grid,),
        in_specs=[
            pl.BlockSpec((BM, 4), lambda i: (i, 0)),
            gspec, gspec, gspec, gspec,
            pl.BlockSpec((4, NUM_BINS), lambda i: (0, 0)),
            pl.BlockSpec((4, D, NUM_BINS), lambda i: (0, 0, 0)),
            pl.BlockSpec((4, D), lambda i: (0, 0)),
            pl.BlockSpec((1, D), lambda i: (0, 0)),
            pl.BlockSpec((1, D), lambda i: (0, 0)),
        ],
        out_specs=pl.BlockSpec((BM, 8, D), lambda i: (i, 0, 0)),
        out_shape=jax.ShapeDtypeStruct((B, 8, D), jnp.float32),
        interpret=interpret,
    )(nums, *gs, centers, w, bias, gamma, beta)


@jax.jit
def kernel(num_0, num_1, num_2, num_3, cat_0, cat_1, cat_2, cat_3,
           centers_0, centers_1, centers_2, centers_3,
           W_0, W_1, W_2, W_3, b_0, b_1, b_2, b_3,
           E_0, E_1, E_2, E_3, gamma, beta):
    sc_gather = _get_sc_gather()
    gs = [sc_gather(E, c.reshape(B // IC, IC))
          for E, c in ((E_0, cat_0), (E_1, cat_1), (E_2, cat_2), (E_3, cat_3))]
    nums = jnp.stack([num_0, num_1, num_2, num_3], axis=1)
    centers = jnp.stack([centers_0, centers_1, centers_2, centers_3])
    w = jnp.stack([W_0, W_1, W_2, W_3])
    bias = jnp.stack([b_0, b_1, b_2, b_3])
    return _tc_call(nums, gs, centers, w, bias, gamma[None, :], beta[None, :])


# speedup vs baseline: 2.1183x; 2.1183x over previous
import functools

import jax
import jax.numpy as jnp
from jax import lax
from jax.experimental import pallas as pl
from jax.experimental.pallas import tpu as pltpu
from jax.experimental.pallas import tpu_sc as plsc

B = 16384
NUM_BINS = 10
D = 64
EPS = 1e-5

NC = 2
NS = 16
NW = NC * NS
BPW = B // NW


@functools.lru_cache(maxsize=None)
def _get_sc_gather():
    mesh = plsc.VectorSubcoreMesh(core_axis_name="c", subcore_axis_name="s",
                                  num_cores=NC, num_subcores=NS)

    @functools.partial(
        pl.kernel,
        mesh=mesh,
        out_type=jax.ShapeDtypeStruct((B, D), jnp.float32),
        scratch_types=[
            pltpu.VMEM((BPW,), jnp.int32),
            pltpu.VMEM((BPW, D), jnp.float32),
            pltpu.SemaphoreType.DMA,
        ],
    )
    def _sc_gather(table, idx, out, idx_v, rows_v, sem):
        wid = lax.axis_index("s") * NC + lax.axis_index("c")
        base = wid * BPW
        pltpu.sync_copy(idx.at[pl.ds(base, BPW)], idx_v)

        def fire(g, carry):
            v = idx_v[pl.ds(g * 16, 16)]
            for l in range(16):
                pltpu.async_copy(table.at[pl.ds(v[l], 1)],
                                 rows_v.at[pl.ds(g * 16 + l, 1)], sem)
            return carry

        lax.fori_loop(0, BPW // 16, fire, 0)

        def drain(j, carry):
            pltpu.make_async_copy(table.at[pl.ds(0, 1)],
                                  rows_v.at[pl.ds(j, 1)], sem).wait()
            return carry

        lax.fori_loop(0, BPW, drain, 0)
        pltpu.sync_copy(rows_v, out.at[pl.ds(base, BPW)])

    return _sc_gather


def _layernorm(t, gamma, beta):
    mu = jnp.mean(t, axis=0, keepdims=True)
    xc = t - mu
    var = jnp.mean(xc * xc, axis=0, keepdims=True)
    return xc * lax.rsqrt(var + EPS) * gamma + beta


def _tc_body(nums_ref, g0_ref, g1_ref, g2_ref, g3_ref, centers_ref, w_ref,
             bias_ref, gamma_ref, beta_ref, out_ref):
    gamma = gamma_ref[...]
    beta = beta_ref[...]
    for f in range(4):
        x = nums_ref[f:f + 1, :]
        mask = jnp.isnan(x)
        clean = jnp.where(mask, 0.0, x)
        d = -((clean - centers_ref[:, f:f + 1]) ** 2)
        d = d - jnp.max(d, axis=0, keepdims=True)
        e = jnp.exp(d)
        p = e / jnp.sum(e, axis=0, keepdims=True)
        tok = jnp.dot(w_ref[f], p, preferred_element_type=jnp.float32)
        tok = tok + bias_ref[:, f:f + 1]
        tok = jnp.where(mask, 0.0, tok)
        out_ref[f] = _layernorm(tok, gamma, beta)
    for f, g_ref in enumerate((g0_ref, g1_ref, g2_ref, g3_ref)):
        t = jnp.transpose(g_ref[...], (1, 0))
        out_ref[4 + f] = _layernorm(t, gamma, beta)


BN = 2048


def _tc_call(nums, gs, centers, w, bias, gamma, beta, interpret=False):
    grid = B // BN
    gspec = pl.BlockSpec((BN, D), lambda i: (i, 0))
    return pl.pallas_call(
        _tc_body,
        grid=(grid,),
        in_specs=[
            pl.BlockSpec((4, BN), lambda i: (0, i)),
            gspec, gspec, gspec, gspec,
            pl.BlockSpec((NUM_BINS, 4), lambda i: (0, 0)),
            pl.BlockSpec((4, D, NUM_BINS), lambda i: (0, 0, 0)),
            pl.BlockSpec((D, 4), lambda i: (0, 0)),
            pl.BlockSpec((D, 1), lambda i: (0, 0)),
            pl.BlockSpec((D, 1), lambda i: (0, 0)),
        ],
        out_specs=pl.BlockSpec((8, D, BN), lambda i: (0, 0, i)),
        out_shape=jax.ShapeDtypeStruct((8, D, B), jnp.float32),
        interpret=interpret,
    )(nums, *gs, centers, w, bias, gamma, beta)


@jax.jit
def kernel(num_0, num_1, num_2, num_3, cat_0, cat_1, cat_2, cat_3,
           centers_0, centers_1, centers_2, centers_3,
           W_0, W_1, W_2, W_3, b_0, b_1, b_2, b_3,
           E_0, E_1, E_2, E_3, gamma, beta):
    sc_gather = _get_sc_gather()
    gs = [sc_gather(E, c) for E, c in
          ((E_0, cat_0), (E_1, cat_1), (E_2, cat_2), (E_3, cat_3))]
    nums = jnp.stack([num_0, num_1, num_2, num_3], axis=0)
    centers = jnp.stack([centers_0, centers_1, centers_2, centers_3], axis=1)
    w = jnp.stack([W_0, W_1, W_2, W_3], axis=0)
    bias = jnp.stack([b_0, b_1, b_2, b_3], axis=1)
    out = _tc_call(nums, gs, centers, w, bias, gamma[:, None], beta[:, None])
    return jnp.transpose(out, (2, 0, 1))
